# baseline (speedup 1.0000x reference)
import functools

import jax
import jax.numpy as jnp
from jax.experimental import pallas as pl
from jax.experimental.pallas import tpu as pltpu

PRE_SEQ_LEN = 128
HIDDEN = 1024
NUM_LAYERS = 24
OUT_DIM = NUM_LAYERS * 2 * HIDDEN
BATCH = 16
BLOCK_N = 2048


def _fused_kernel(prefix_ref, emb_ref, w1_ref, b1_ref, w2_ref, b2_ref,
                  out_ref, htab_ref, onehot_ref):
    j = pl.program_id(0)

    @pl.when(j == 0)
    def _init():
        h = jnp.dot(emb_ref[...], w1_ref[...],
                    preferred_element_type=jnp.float32)
        htab_ref[...] = jnp.tanh(h + b1_ref[...]).astype(jnp.bfloat16)
        ids = prefix_ref[...]
        iota = jax.lax.broadcasted_iota(jnp.int32, (BATCH, PRE_SEQ_LEN, PRE_SEQ_LEN), 2)
        onehot_ref[...] = (ids[:, :, None] == iota).astype(jnp.bfloat16)

    t = jnp.dot(htab_ref[...], w2_ref[...],
                preferred_element_type=jnp.float32).astype(jnp.bfloat16)
    out_ref[...] = jax.lax.dot_general(
        onehot_ref[...], t,
        dimension_numbers=(((2,), (0,)), ((), ())),
        preferred_element_type=jnp.float32) + b2_ref[...]


@jax.jit
def kernel(prefix, emb, W1, b1, W2, b2):
    prefix = prefix.astype(jnp.int32)
    W2 = W2.astype(jnp.bfloat16)
    b1r = b1.reshape(1, HIDDEN)
    b2r = b2.reshape(1, OUT_DIM)
    grid = (OUT_DIM // BLOCK_N,)
    out = pl.pallas_call(
        _fused_kernel,
        grid=grid,
        in_specs=[
            pl.BlockSpec((BATCH, PRE_SEQ_LEN), lambda j: (0, 0)),
            pl.BlockSpec((PRE_SEQ_LEN, HIDDEN), lambda j: (0, 0)),
            pl.BlockSpec((HIDDEN, HIDDEN), lambda j: (0, 0)),
            pl.BlockSpec((1, HIDDEN), lambda j: (0, 0)),
            pl.BlockSpec((HIDDEN, BLOCK_N), lambda j: (0, j)),
            pl.BlockSpec((1, BLOCK_N), lambda j: (0, j)),
        ],
        out_specs=pl.BlockSpec((BATCH, PRE_SEQ_LEN, BLOCK_N), lambda j: (0, 0, j)),
        out_shape=jax.ShapeDtypeStruct((BATCH, PRE_SEQ_LEN, OUT_DIM), jnp.float32),
        scratch_shapes=[
            pltpu.VMEM((PRE_SEQ_LEN, HIDDEN), jnp.bfloat16),
            pltpu.VMEM((BATCH, PRE_SEQ_LEN, PRE_SEQ_LEN), jnp.bfloat16),
        ],
        compiler_params=pltpu.CompilerParams(
            dimension_semantics=("arbitrary",),
        ),
    )(prefix, emb, W1, b1r, W2, b2r)
    return out


# speedup vs baseline: 1.3049x; 1.3049x over previous
import functools

import jax
import jax.numpy as jnp
from jax.experimental import pallas as pl
from jax.experimental.pallas import tpu as pltpu

PRE_SEQ_LEN = 128
HIDDEN = 1024
NUM_LAYERS = 24
OUT_DIM = NUM_LAYERS * 2 * HIDDEN
BATCH = 16
BLOCK_N = 2048


def _fused_kernel(prefix_ref, emb_ref, w1_ref, b1_ref, w2_ref, b2_ref,
                  out_ref, htab_ref, onehot_ref):
    j = pl.program_id(0)

    @pl.when(j == 0)
    def _init():
        h = jnp.dot(emb_ref[...], w1_ref[...],
                    preferred_element_type=jnp.float32)
        htab_ref[...] = jnp.tanh(h + b1_ref[...]).astype(jnp.bfloat16)
        ids = prefix_ref[...]
        iota = jax.lax.broadcasted_iota(jnp.int32, (BATCH, PRE_SEQ_LEN, PRE_SEQ_LEN), 2)
        onehot_ref[...] = (ids[:, :, None] == iota).astype(jnp.bfloat16)

    t = jnp.dot(htab_ref[...], w2_ref[...].astype(jnp.bfloat16),
                preferred_element_type=jnp.float32).astype(jnp.bfloat16)
    out_ref[...] = jax.lax.dot_general(
        onehot_ref[...], t,
        dimension_numbers=(((2,), (0,)), ((), ())),
        preferred_element_type=jnp.float32) + b2_ref[...]


@jax.jit
def kernel(prefix, emb, W1, b1, W2, b2):
    prefix = prefix.astype(jnp.int32)
    b1r = b1.reshape(1, HIDDEN)
    b2r = b2.reshape(1, OUT_DIM)
    grid = (OUT_DIM // BLOCK_N,)
    out = pl.pallas_call(
        _fused_kernel,
        grid=grid,
        in_specs=[
            pl.BlockSpec((BATCH, PRE_SEQ_LEN), lambda j: (0, 0)),
            pl.BlockSpec((PRE_SEQ_LEN, HIDDEN), lambda j: (0, 0)),
            pl.BlockSpec((HIDDEN, HIDDEN), lambda j: (0, 0)),
            pl.BlockSpec((1, HIDDEN), lambda j: (0, 0)),
            pl.BlockSpec((HIDDEN, BLOCK_N), lambda j: (0, j)),
            pl.BlockSpec((1, BLOCK_N), lambda j: (0, j)),
        ],
        out_specs=pl.BlockSpec((BATCH, PRE_SEQ_LEN, BLOCK_N), lambda j: (0, 0, j)),
        out_shape=jax.ShapeDtypeStruct((BATCH, PRE_SEQ_LEN, OUT_DIM), jnp.float32),
        scratch_shapes=[
            pltpu.VMEM((PRE_SEQ_LEN, HIDDEN), jnp.bfloat16),
            pltpu.VMEM((BATCH, PRE_SEQ_LEN, PRE_SEQ_LEN), jnp.bfloat16),
        ],
        compiler_params=pltpu.CompilerParams(
            dimension_semantics=("arbitrary",),
        ),
    )(prefix, emb, W1, b1r, W2, b2r)
    return out
